# baseline (speedup 1.0000x reference)
import functools

import jax
import jax.numpy as jnp
from jax import lax
from jax.experimental import pallas as pl
from jax.experimental.pallas import tpu as pltpu
from jax.experimental.pallas import tpu_sc as plsc

NC = 2
NS = 16
NW = NC * NS

CHUNK = 2048
SUB = 2048


def _make_lookup(total: int, D: int):
  per_tile = total // NW
  n_chunks = per_tile // CHUNK
  assert n_chunks % 2 == 0 and n_chunks >= 4
  n_pairs = n_chunks // 2
  mesh = plsc.VectorSubcoreMesh(core_axis_name="c", subcore_axis_name="s")

  @functools.partial(
      pl.kernel,
      out_type=jax.ShapeDtypeStruct((total, D), jnp.float32),
      mesh=mesh,
      scratch_types=[
          pltpu.VMEM((CHUNK,), jnp.int32),
          pltpu.VMEM((CHUNK,), jnp.int32),
          pltpu.VMEM((CHUNK, D), jnp.float32),
          pltpu.VMEM((CHUNK, D), jnp.float32),
          pltpu.SemaphoreType.DMA,
          pltpu.SemaphoreType.DMA,
          pltpu.SemaphoreType.DMA,
          pltpu.SemaphoreType.DMA,
          pltpu.SemaphoreType.DMA,
          pltpu.SemaphoreType.DMA,
      ],
      compiler_params=pltpu.CompilerParams(use_tc_tiling_on_sc=False),
  )
  def lookup(x_hbm, w_hbm, out_hbm, idx0, idx1, rows0, rows1,
             si0, si1, sg0, sg1, so0, so1):
    wid = lax.axis_index("s") * NC + lax.axis_index("c")
    base = wid * per_tile
    idx_b = (idx0, idx1)
    rows_b = (rows0, rows1)
    si = (si0, si1)
    sg = (sg0, sg1)
    so = (so0, so1)

    def idx_copy(g, slot):
      off = base + g * CHUNK
      pltpu.async_copy(x_hbm.at[pl.ds(off, CHUNK)], idx_b[slot], si[slot])

    idx_copy(0, 0)
    idx_copy(1, 1)

    def pair(p, carry):
      for slot in (0, 1):
        g = 2 * p + slot
        off = base + g * CHUNK

        @pl.when(p > 0)
        def _():
          pltpu.make_async_copy(
              rows_b[slot], out_hbm.at[pl.ds(base, CHUNK)], so[slot]).wait()

        pltpu.make_async_copy(
            x_hbm.at[pl.ds(off, CHUNK)], idx_b[slot], si[slot]).wait()

        gathers = [
            pltpu.async_copy(
                w_hbm.at[idx_b[slot].at[pl.ds(j * SUB, SUB)]],
                rows_b[slot].at[pl.ds(j * SUB, SUB)],
                sg[slot],
            )
            for j in range(CHUNK // SUB)
        ]

        for cp in gathers:
          cp.wait()

        @pl.when(g + 2 < n_chunks)
        def _():
          idx_copy(g + 2, slot)

        pltpu.async_copy(rows_b[slot], out_hbm.at[pl.ds(off, CHUNK)], so[slot])
      return carry

    lax.fori_loop(0, n_pairs, pair, 0)

    for slot in (0, 1):
      pltpu.make_async_copy(
          rows_b[slot], out_hbm.at[pl.ds(base, CHUNK)], so[slot]).wait()

  return lookup


def kernel(x, W):
  B, L = x.shape
  N, D = W.shape
  total = B * L
  x_flat = x.reshape(total).astype(jnp.int32)
  out = _make_lookup(total, D)(x_flat, W)
  return out.reshape(B, L, D)


# speedup vs baseline: 1.2745x; 1.2745x over previous
import functools

import jax
import jax.numpy as jnp
from jax import lax
from jax.experimental import pallas as pl
from jax.experimental.pallas import tpu as pltpu
from jax.experimental.pallas import tpu_sc as plsc

NC = 2
NS = 16
NW = NC * NS

CHUNK = 1024


def _make_lookup(L: int, B: int, N: int, D: int):
  spans = B // CHUNK
  n_items = L * spans
  per_tile = n_items // NW
  assert per_tile % 2 == 0 and per_tile >= 4
  n_pairs = per_tile // 2
  mesh = plsc.VectorSubcoreMesh(core_axis_name="c", subcore_axis_name="s")

  @functools.partial(
      pl.kernel,
      out_type=jax.ShapeDtypeStruct((L * D, B), jnp.float32),
      mesh=mesh,
      scratch_types=[
          pltpu.VMEM((CHUNK,), jnp.int32),
          pltpu.VMEM((CHUNK,), jnp.int32),
          pltpu.VMEM((CHUNK, D), jnp.float32),
          pltpu.VMEM((CHUNK, D), jnp.float32),
          pltpu.VMEM((D, CHUNK), jnp.float32),
          pltpu.VMEM((D, CHUNK), jnp.float32),
          pltpu.SemaphoreType.DMA,
          pltpu.SemaphoreType.DMA,
          pltpu.SemaphoreType.DMA,
          pltpu.SemaphoreType.DMA,
          pltpu.SemaphoreType.DMA,
          pltpu.SemaphoreType.DMA,
      ],
      compiler_params=pltpu.CompilerParams(
          use_tc_tiling_on_sc=False, needs_layout_passes=False),
  )
  def lookup(xt_hbm, w_hbm, out_hbm, idx0, idx1, rows0, rows1, t0, t1,
             si0, si1, sg0, sg1, so0, so1):
    wid = lax.axis_index("s") * NC + lax.axis_index("c")
    item0 = wid * per_tile
    idx_b = (idx0, idx1)
    rows_b = (rows0, rows1)
    t_b = (t0, t1)
    si = (si0, si1)
    sg = (sg0, sg1)
    so = (so0, so1)
    lane = lax.iota(jnp.int32, D)

    def idx_copy(g, slot):
      item = item0 + g
      l = item // spans
      b0 = (item % spans) * CHUNK
      pltpu.async_copy(xt_hbm.at[l, pl.ds(b0, CHUNK)], idx_b[slot], si[slot])

    idx_copy(0, 0)
    idx_copy(1, 1)

    def pair(p, carry):
      for slot in (0, 1):
        g = 2 * p + slot
        item = item0 + g
        l = item // spans
        b0 = (item % spans) * CHUNK

        pltpu.make_async_copy(
            xt_hbm.at[l, pl.ds(b0, CHUNK)], idx_b[slot], si[slot]).wait()
        gcp = pltpu.async_copy(
            w_hbm.at[idx_b[slot]], rows_b[slot], sg[slot])

        @pl.when(p > 0)
        def _():
          pltpu.make_async_copy(
              t_b[slot], out_hbm.at[pl.ds(0, D), pl.ds(0, CHUNK)],
              so[slot]).wait()

        gcp.wait()

        @pl.when(g + 2 < per_tile)
        def _():
          idx_copy(g + 2, slot)

        rows = rows_b[slot]
        t = t_b[slot]

        @pl.loop(0, CHUNK, step=8)
        def _(i0):
          for j in range(8):
            i = i0 + j
            v = rows[i, :]
            plsc.store_scatter(t, [lane, jnp.full((D,), i, jnp.int32)], v)

        pltpu.async_copy(
            t, out_hbm.at[pl.ds(l * D, D), pl.ds(b0, CHUNK)], so[slot])
      return carry

    lax.fori_loop(0, n_pairs, pair, 0)

    for slot in (0, 1):
      pltpu.make_async_copy(
          t_b[slot], out_hbm.at[pl.ds(0, D), pl.ds(0, CHUNK)],
          so[slot]).wait()

  return lookup


def kernel(x, W):
  B, L = x.shape
  N, D = W.shape
  xt = x.T.astype(jnp.int32)
  out2d = _make_lookup(L, B, N, D)(xt, W)
  return jnp.transpose(out2d.reshape(L, D, B), (2, 0, 1))


# speedup vs baseline: 1.8229x; 1.4303x over previous
import functools

import jax
import jax.numpy as jnp
from jax import lax
from jax.experimental import pallas as pl
from jax.experimental.pallas import tpu as pltpu
from jax.experimental.pallas import tpu_sc as plsc

NC = 2
NS = 16
NW = NC * NS

CHUNK = 1024


def _make_lookup(L: int, B: int, N: int, D: int):
  spans = B // CHUNK
  n_items = L * spans
  per_tile = n_items // NW
  assert per_tile % 2 == 0 and per_tile >= 4
  n_pairs = per_tile // 2
  mesh = plsc.VectorSubcoreMesh(core_axis_name="c", subcore_axis_name="s")

  @functools.partial(
      pl.kernel,
      out_type=jax.ShapeDtypeStruct((L * D, B), jnp.float32),
      mesh=mesh,
      scratch_types=[
          pltpu.VMEM((CHUNK,), jnp.int32),
          pltpu.VMEM((CHUNK,), jnp.int32),
          pltpu.VMEM((CHUNK, D), jnp.float32),
          pltpu.VMEM((CHUNK, D), jnp.float32),
          pltpu.VMEM((D, CHUNK), jnp.float32),
          pltpu.VMEM((D, CHUNK), jnp.float32),
          pltpu.SemaphoreType.DMA,
          pltpu.SemaphoreType.DMA,
          pltpu.SemaphoreType.DMA,
          pltpu.SemaphoreType.DMA,
          pltpu.SemaphoreType.DMA,
          pltpu.SemaphoreType.DMA,
      ],
      compiler_params=pltpu.CompilerParams(
          use_tc_tiling_on_sc=False, needs_layout_passes=False),
  )
  def lookup(xt_hbm, w_hbm, out_hbm, idx0, idx1, rows0, rows1, t0, t1,
             si0, si1, sg0, sg1, so0, so1):
    wid = lax.axis_index("s") * NC + lax.axis_index("c")
    item0 = wid * per_tile
    idx_b = (idx0, idx1)
    rows_b = (rows0, rows1)
    t_b = (t0, t1)
    si = (si0, si1)
    sg = (sg0, sg1)
    so = (so0, so1)
    lane = lax.iota(jnp.int32, D)
    rots = [(lane + s) % 16 for s in range(16)]

    def idx_copy(g, slot):
      item = item0 + g
      l = item // spans
      b0 = (item % spans) * CHUNK
      pltpu.async_copy(xt_hbm.at[l, pl.ds(b0, CHUNK)], idx_b[slot], si[slot])

    idx_copy(0, 0)
    idx_copy(1, 1)

    def pair(p, carry):
      for slot in (0, 1):
        g = 2 * p + slot
        item = item0 + g
        l = item // spans
        b0 = (item % spans) * CHUNK

        pltpu.make_async_copy(
            xt_hbm.at[l, pl.ds(b0, CHUNK)], idx_b[slot], si[slot]).wait()
        gcp = pltpu.async_copy(
            w_hbm.at[idx_b[slot]], rows_b[slot], sg[slot])

        @pl.when(p > 0)
        def _():
          pltpu.make_async_copy(
              t_b[slot], out_hbm.at[pl.ds(0, D), pl.ds(0, CHUNK)],
              so[slot]).wait()

        gcp.wait()

        @pl.when(g + 2 < per_tile)
        def _():
          idx_copy(g + 2, slot)

        rows = rows_b[slot]
        t = t_b[slot]

        @pl.loop(0, CHUNK, step=16)
        def _(i0):
          for s in range(16):
            ri = rots[s] + i0
            v = plsc.load_gather(rows, [ri, lane])
            plsc.store_scatter(t, [lane, ri], v)

        pltpu.async_copy(
            t, out_hbm.at[pl.ds(l * D, D), pl.ds(b0, CHUNK)], so[slot])
      return carry

    lax.fori_loop(0, n_pairs, pair, 0)

    for slot in (0, 1):
      pltpu.make_async_copy(
          t_b[slot], out_hbm.at[pl.ds(0, D), pl.ds(0, CHUNK)],
          so[slot]).wait()

  return lookup


def kernel(x, W):
  B, L = x.shape
  N, D = W.shape
  xt = x.T.astype(jnp.int32)
  out2d = _make_lookup(L, B, N, D)(xt, W)
  return jnp.transpose(out2d.reshape(L, D, B), (2, 0, 1))


# speedup vs baseline: 2.2392x; 1.2283x over previous
import functools

import jax
import jax.numpy as jnp
from jax import lax
from jax.experimental import pallas as pl
from jax.experimental.pallas import tpu as pltpu
from jax.experimental.pallas import tpu_sc as plsc

NC = 2
NS = 16
NW = NC * NS

CHUNK = 1024


def _make_lookup(L: int, B: int, N: int, D: int):
  spans = B // CHUNK
  n_items = L * spans
  per_tile = n_items // NW
  assert per_tile % 2 == 0 and per_tile >= 4
  n_pairs = per_tile // 2
  mesh = plsc.VectorSubcoreMesh(core_axis_name="c", subcore_axis_name="s")

  @functools.partial(
      pl.kernel,
      out_type=jax.ShapeDtypeStruct((L * D, B), jnp.float32),
      mesh=mesh,
      scratch_types=[
          pltpu.VMEM((CHUNK,), jnp.int32),
          pltpu.VMEM((CHUNK,), jnp.int32),
          pltpu.VMEM((CHUNK, D), jnp.float32),
          pltpu.VMEM((CHUNK, D), jnp.float32),
          pltpu.VMEM((D, CHUNK), jnp.float32),
          pltpu.VMEM((D, CHUNK), jnp.float32),
          pltpu.SemaphoreType.DMA,
          pltpu.SemaphoreType.DMA,
          pltpu.SemaphoreType.DMA,
          pltpu.SemaphoreType.DMA,
          pltpu.SemaphoreType.DMA,
          pltpu.SemaphoreType.DMA,
      ],
      compiler_params=pltpu.CompilerParams(
          use_tc_tiling_on_sc=False, needs_layout_passes=False),
  )
  def lookup(xt_hbm, w_hbm, out_hbm, idx0, idx1, rows0, rows1, t0, t1,
             si0, si1, sg0, sg1, so0, so1):
    wid = lax.axis_index("s") * NC + lax.axis_index("c")
    item0 = wid * per_tile
    idx_b = (idx0, idx1)
    rows_b = (rows0, rows1)
    t_b = (t0, t1)
    si = (si0, si1)
    sg = (sg0, sg1)
    so = (so0, so1)
    lane = lax.iota(jnp.int32, D)
    rots = [(lane + s) % 16 for s in range(16)]

    def idx_copy(g, slot):
      item = item0 + g
      l = item // spans
      b0 = (item % spans) * CHUNK
      pltpu.async_copy(xt_hbm.at[l, pl.ds(b0, CHUNK)], idx_b[slot], si[slot])

    def fire_gather(slot):
      return pltpu.async_copy(w_hbm.at[idx_b[slot]], rows_b[slot], sg[slot])

    idx_copy(0, 0)
    idx_copy(1, 1)
    pltpu.make_async_copy(
        xt_hbm.at[0, pl.ds(0, CHUNK)], idx_b[0], si[0]).wait()
    fire_gather(0)

    def pair(p, carry):
      for slot in (0, 1):
        g = 2 * p + slot
        other = 1 - slot
        item = item0 + g
        l = item // spans
        b0 = (item % spans) * CHUNK

        pltpu.make_async_copy(
            w_hbm.at[idx_b[slot]], rows_b[slot], sg[slot]).wait()

        @pl.when(g + 2 < per_tile)
        def _():
          idx_copy(g + 2, slot)

        @pl.when(g + 1 < per_tile)
        def _():
          pltpu.make_async_copy(
              xt_hbm.at[0, pl.ds(0, CHUNK)], idx_b[other], si[other]).wait()
          fire_gather(other)

        @pl.when(p > 0)
        def _():
          pltpu.make_async_copy(
              t_b[slot], out_hbm.at[pl.ds(0, D), pl.ds(0, CHUNK)],
              so[slot]).wait()

        rows = rows_b[slot]
        t = t_b[slot]

        @pl.loop(0, CHUNK, step=16)
        def _(i0):
          for s in range(16):
            ri = rots[s] + i0
            v = plsc.load_gather(rows, [ri, lane])
            plsc.store_scatter(t, [lane, ri], v)

        pltpu.async_copy(
            t, out_hbm.at[pl.ds(l * D, D), pl.ds(b0, CHUNK)], so[slot])
      return carry

    lax.fori_loop(0, n_pairs, pair, 0)

    for slot in (0, 1):
      pltpu.make_async_copy(
          t_b[slot], out_hbm.at[pl.ds(0, D), pl.ds(0, CHUNK)],
          so[slot]).wait()

  return lookup


def kernel(x, W):
  B, L = x.shape
  N, D = W.shape
  xt = x.T.astype(jnp.int32)
  out2d = _make_lookup(L, B, N, D)(xt, W)
  return jnp.transpose(out2d.reshape(L, D, B), (2, 0, 1))
